# baseline (speedup 1.0000x reference)
import functools

import jax
import jax.numpy as jnp
from jax import lax
from jax.experimental import pallas as pl
from jax.experimental.pallas import tpu as pltpu
from jax.experimental.pallas import tpu_sc as plsc

B = 4
L = 512
H = 12
MAXS = 512
TAB = 2 * MAXS

NC = 2
NS = 16
LANES = 16
NW = NC * NS
CHUNKS = (B * L) // LANES
CPW = CHUNKS // NW
WPB = NW // B
JB = 128
NJB = L // JB
NSLOT = 4
GROUPS = CPW * NJB

HP = H // 2
RS = HP + 1
TW = TAB * RS


def _project_body(pe_ref, w_ref, b_ref, o_ref):
    for k in range(4):
        t = jnp.dot(pe_ref[k], w_ref[k], preferred_element_type=jnp.float32)
        if k == 0:
            t = t + b_ref[:]
        o_ref[k] = t


_project = pl.pallas_call(
    _project_body,
    out_shape=jax.ShapeDtypeStruct((4, TAB, H), jnp.float32),
)


@functools.partial(
    pl.kernel,
    mesh=plsc.VectorSubcoreMesh(core_axis_name="c", subcore_axis_name="s"),
    out_type=jax.ShapeDtypeStruct((B, H, L, L), jnp.float32),
    compiler_params=pltpu.CompilerParams(needs_layout_passes=False),
    scratch_types=[
        pltpu.VMEM((4 * TW,), jnp.int32),
        pltpu.VMEM((L,), jnp.int32),
        pltpu.VMEM((L,), jnp.int32),
        pltpu.VMEM((H, LANES, JB), jnp.float32),
        pltpu.VMEM((H, LANES, JB), jnp.float32),
        pltpu.VMEM((H, LANES, JB), jnp.float32),
        pltpu.VMEM((H, LANES, JB), jnp.float32),
        pltpu.SemaphoreType.DMA,
        pltpu.SemaphoreType.DMA,
        pltpu.SemaphoreType.DMA,
        pltpu.SemaphoreType.DMA,
    ],
)
def _sc_fuse(t_hbm, ps_hbm, pe_hbm, out_hbm, t_vm, ps_vm, pe_vm,
             buf0, buf1, buf2, buf3, sem0, sem1, sem2, sem3):
    wid = lax.axis_index("s") * NC + lax.axis_index("c")
    bidx = wid // WPB
    pltpu.sync_copy(t_hbm, t_vm)
    pltpu.sync_copy(ps_hbm.at[bidx], ps_vm)
    pltpu.sync_copy(pe_hbm.at[bidx], pe_vm)

    lane = lax.iota(jnp.int32, LANES)
    bufs = (buf0, buf1, buf2, buf3)
    sems = (sem0, sem1, sem2, sem3)

    def group_body(g, _):
        for pb in range(NSLOT):
            gi = g * NSLOT + pb
            c = gi // NJB
            jb = gi % NJB
            i0 = ((wid % WPB) * CPW + c) * LANES
            j0 = jb * JB
            buf = bufs[pb]
            sem = sems[pb]

            @pl.when(g > 0)
            def _drain():
                pltpu.make_async_copy(
                    buf,
                    out_hbm.at[0, :, pl.ds(0, LANES), pl.ds(0, JB)],
                    sem,
                ).wait()

            ivec = jnp.full((LANES,), i0, jnp.int32) + lane
            vs = plsc.load_gather(ps_vm, [ivec])
            ve = plsc.load_gather(pe_vm, [ivec])
            vs_b = (vs + MAXS) * RS
            ve_b = (ve + MAXS) * RS + 2 * TW

            @plsc.parallel_loop(0, JB, 1, unroll=2)
            def jbody(jj, vs_b=vs_b, ve_b=ve_b, buf=buf, j0=j0):
                j = j0 + jj
                psj = plsc.load_gather(ps_vm, [jnp.full((LANES,), j, jnp.int32)])
                pej = plsc.load_gather(pe_vm, [jnp.full((LANES,), j, jnp.int32)])
                psj_h = psj * RS
                pej_h = pej * RS - TW
                i_ss = vs_b - psj_h
                i_se = vs_b - pej_h
                i_es = ve_b - psj_h
                i_ee = ve_b - pej_h
                jjv = jnp.full((LANES,), jj, jnp.int32)
                for hp in range(HP):
                    a1 = plsc.bitcast(plsc.load_gather(t_vm, [i_ss + hp]), jnp.bfloat16)
                    a2 = plsc.bitcast(plsc.load_gather(t_vm, [i_se + hp]), jnp.bfloat16)
                    a3 = plsc.bitcast(plsc.load_gather(t_vm, [i_es + hp]), jnp.bfloat16)
                    a4 = plsc.bitcast(plsc.load_gather(t_vm, [i_ee + hp]), jnp.bfloat16)
                    s = (a1 + a2) + (a3 + a4)
                    s = jnp.maximum(s, jnp.bfloat16(0))
                    v_even, v_odd = plsc.unpack(s, format=plsc.PackFormat.INTERLEAVED)
                    plsc.store_scatter(
                        buf,
                        [jnp.full((LANES,), 2 * hp, jnp.int32), lane, jjv],
                        v_even,
                    )
                    plsc.store_scatter(
                        buf,
                        [jnp.full((LANES,), 2 * hp + 1, jnp.int32), lane, jjv],
                        v_odd,
                    )

            pltpu.async_copy(
                buf,
                out_hbm.at[bidx, :, pl.ds(i0, LANES), pl.ds(j0, JB)],
                sem,
            )
        return 0

    lax.fori_loop(0, GROUPS // NSLOT, group_body, 0)
    for pb in range(NSLOT):
        pltpu.make_async_copy(
            bufs[pb],
            out_hbm.at[0, :, pl.ds(0, LANES), pl.ds(0, JB)],
            sems[pb],
        ).wait()


def kernel(pos_s, pos_e, pe_ss, pe_se, pe_es, pe_ee, W, b):
    ps = pos_s.astype(jnp.int32)
    pe = pos_e.astype(jnp.int32)
    tables = jnp.stack([pe_ss, pe_se, pe_es, pe_ee])
    wr = W.reshape(4, H, H)
    t = _project(tables, wr, b.reshape(1, H))
    t_pk = jax.lax.bitcast_convert_type(
        t.astype(jnp.bfloat16).reshape(4, TAB, HP, 2), jnp.int32)
    t_pk = jnp.pad(t_pk, ((0, 0), (0, 0), (0, RS - HP))).reshape(4 * TW)
    return _sc_fuse(t_pk, ps, pe)


# speedup vs baseline: 3.1198x; 3.1198x over previous
import functools

import jax
import jax.numpy as jnp
from jax import lax
from jax.experimental import pallas as pl
from jax.experimental.pallas import tpu as pltpu
from jax.experimental.pallas import tpu_sc as plsc

B = 4
L = 512
H = 12
MAXS = 512
TAB = 2 * MAXS

NC = 2
NS = 16
LANES = 16
NW = NC * NS
WPB = NW // B
RPW = L // WPB
NSLOT = 4

HP = H // 2
RS = HP + 1
TW = TAB * RS


def _project_body(pe_ref, w_ref, b_ref, o_ref):
    for k in range(4):
        t = jnp.dot(pe_ref[k], w_ref[k], preferred_element_type=jnp.float32)
        if k == 0:
            t = t + b_ref[:]
        o_ref[k] = t


_project = pl.pallas_call(
    _project_body,
    out_shape=jax.ShapeDtypeStruct((4, TAB, H), jnp.float32),
)


@functools.partial(
    pl.kernel,
    mesh=plsc.VectorSubcoreMesh(core_axis_name="c", subcore_axis_name="s"),
    out_type=jax.ShapeDtypeStruct((B, H, L, L), jnp.float32),
    compiler_params=pltpu.CompilerParams(needs_layout_passes=False),
    scratch_types=[
        pltpu.VMEM((4 * TW,), jnp.int32),
        pltpu.VMEM((L,), jnp.int32),
        pltpu.VMEM((L,), jnp.int32),
        pltpu.VMEM((H, L), jnp.float32),
        pltpu.VMEM((H, L), jnp.float32),
        pltpu.VMEM((H, L), jnp.float32),
        pltpu.VMEM((H, L), jnp.float32),
        pltpu.SemaphoreType.DMA,
        pltpu.SemaphoreType.DMA,
        pltpu.SemaphoreType.DMA,
        pltpu.SemaphoreType.DMA,
    ],
)
def _sc_fuse(t_hbm, ps_hbm, pe_hbm, out_hbm, t_vm, ps_vm, pe_vm,
             buf0, buf1, buf2, buf3, sem0, sem1, sem2, sem3):
    wid = lax.axis_index("s") * NC + lax.axis_index("c")
    bidx = wid // WPB
    pltpu.sync_copy(t_hbm, t_vm)
    pltpu.sync_copy(ps_hbm.at[bidx], ps_vm)
    pltpu.sync_copy(pe_hbm.at[bidx], pe_vm)

    lane = lax.iota(jnp.int32, LANES)
    bufs = (buf0, buf1, buf2, buf3)
    sems = (sem0, sem1, sem2, sem3)

    def group_body(g, _):
        for pb in range(NSLOT):
            r = g * NSLOT + pb
            i_loc = (wid % WPB) * RPW + r
            buf = bufs[pb]
            sem = sems[pb]

            @pl.when(g > 0)
            def _drain():
                pltpu.make_async_copy(
                    buf,
                    out_hbm.at[0, :, 0, :],
                    sem,
                ).wait()

            siv = plsc.load_gather(ps_vm, [jnp.full((LANES,), i_loc, jnp.int32)])
            eiv = plsc.load_gather(pe_vm, [jnp.full((LANES,), i_loc, jnp.int32)])
            a_s = (siv + MAXS) * RS
            a_e = (eiv + MAXS) * RS + 2 * TW

            @plsc.parallel_loop(0, L, LANES, unroll=2)
            def jv_body(jv, a_s=a_s, a_e=a_e, buf=buf):
                jvec = jnp.full((LANES,), jv, jnp.int32) + lane
                psj = plsc.load_gather(ps_vm, [jvec])
                pej = plsc.load_gather(pe_vm, [jvec])
                psj_r = psj * RS
                pej_r = pej * RS - TW
                i_ss = a_s - psj_r
                i_se = a_s - pej_r
                i_es = a_e - psj_r
                i_ee = a_e - pej_r
                for hp in range(HP):
                    a1 = plsc.bitcast(plsc.load_gather(t_vm, [i_ss + hp]), jnp.bfloat16)
                    a2 = plsc.bitcast(plsc.load_gather(t_vm, [i_se + hp]), jnp.bfloat16)
                    a3 = plsc.bitcast(plsc.load_gather(t_vm, [i_es + hp]), jnp.bfloat16)
                    a4 = plsc.bitcast(plsc.load_gather(t_vm, [i_ee + hp]), jnp.bfloat16)
                    s = (a1 + a2) + (a3 + a4)
                    s = jnp.maximum(s, jnp.bfloat16(0))
                    v_even, v_odd = plsc.unpack(s, format=plsc.PackFormat.INTERLEAVED)
                    plsc.store_scatter(
                        buf, [jnp.full((LANES,), 2 * hp, jnp.int32), jvec], v_even)
                    plsc.store_scatter(
                        buf, [jnp.full((LANES,), 2 * hp + 1, jnp.int32), jvec], v_odd)

            pltpu.async_copy(
                buf,
                out_hbm.at[bidx, :, i_loc, :],
                sem,
            )
        return 0

    lax.fori_loop(0, RPW // NSLOT, group_body, 0)
    for pb in range(NSLOT):
        pltpu.make_async_copy(
            bufs[pb],
            out_hbm.at[0, :, 0, :],
            sems[pb],
        ).wait()


def kernel(pos_s, pos_e, pe_ss, pe_se, pe_es, pe_ee, W, b):
    ps = pos_s.astype(jnp.int32)
    pe = pos_e.astype(jnp.int32)
    tables = jnp.stack([pe_ss, pe_se, pe_es, pe_ee])
    wr = W.reshape(4, H, H)
    t = _project(tables, wr, b.reshape(1, H))
    t_pk = jax.lax.bitcast_convert_type(
        t.astype(jnp.bfloat16).reshape(4, TAB, HP, 2), jnp.int32)
    t_pk = jnp.pad(t_pk, ((0, 0), (0, 0), (0, RS - HP))).reshape(4 * TW)
    return _sc_fuse(t_pk, ps, pe)
